# baseline (speedup 1.0000x reference)
import functools

import jax
import jax.numpy as jnp
from jax import lax
from jax.experimental import pallas as pl
from jax.experimental.pallas import tpu as pltpu
from jax.experimental.pallas import tpu_sc as plsc

B_ = 16
D_ = 256
T_ = 4096
NUM_CODES = 8192
MARGIN = 0.2
STRIDE = 320

L_ = 16
NW_ = 32
C_ = 128
PER_W = (B_ * T_) // NW_
NCHUNK_ = PER_W // C_
G_ = C_ // L_
ACC_ = 4


def _sc_body(s_hbm, codes_hbm, neg_hbm, cb_hbm, posq_hbm, negq_hbm,
             idxp_v, idxn_v, s_v, rowsp_v, rowsn_v, row0_v, outp_v, outn_v,
             semp, semn, sems):
    cid = lax.axis_index("c")
    sid = lax.axis_index("s")
    wid = sid * 2 + cid
    b = wid // 2
    h = wid % 2
    t_base = h * (T_ // 2)
    pltpu.sync_copy(cb_hbm.at[0], row0_v)
    lane = lax.iota(jnp.int32, L_)
    zeros = jnp.zeros((L_,), jnp.float32)

    def chunk_body(ci, carry):
        t0 = t_base + ci * C_
        pltpu.sync_copy(codes_hbm.at[b, pl.ds(t0, C_)], idxp_v)
        pltpu.sync_copy(neg_hbm.at[b, pl.ds(t0, C_)], idxn_v)
        cp_p = pltpu.async_copy(cb_hbm.at[idxp_v], rowsp_v, semp)
        cp_n = pltpu.async_copy(cb_hbm.at[idxn_v], rowsn_v, semn)
        cp_s = pltpu.async_copy(s_hbm.at[b, :, pl.ds(t0, C_)], s_v, sems)
        cp_p.wait()
        cp_n.wait()
        cp_s.wait()
        for g in range(G_):
            row_idx = lane + g * L_
            edge = g == 0 or g == G_ - 1
            if edge:
                def dbody3(j, acc):
                    aps, ans, a0s = acc
                    aps, ans, a0s = list(aps), list(ans), list(a0s)
                    base = j * ACC_
                    for k in range(ACC_):
                        dvec = (lane + (base + k)) & (D_ - 1)
                        sv = plsc.load_gather(s_v, [dvec, row_idx])
                        pv = plsc.load_gather(rowsp_v, [row_idx, dvec])
                        nv = plsc.load_gather(rowsn_v, [row_idx, dvec])
                        c0 = plsc.load_gather(row0_v, [dvec])
                        dp = sv - pv
                        dn = sv - nv
                        d0 = sv - c0
                        aps[k] = aps[k] + dp * dp
                        ans[k] = ans[k] + dn * dn
                        a0s[k] = a0s[k] + d0 * d0
                    return tuple(aps), tuple(ans), tuple(a0s)

                aps, ans, a0s = lax.fori_loop(
                    0, D_ // ACC_, dbody3,
                    ((zeros,) * ACC_, (zeros,) * ACC_, (zeros,) * ACC_))
                ap = sum(aps[1:], aps[0])
                an = sum(ans[1:], ans[0])
                a0 = sum(a0s[1:], a0s[0])
                if g == 0:
                    cond = (h == 0) & (ci == 0) & (lane == 0)
                else:
                    cond = (h == 1) & (ci == NCHUNK_ - 1) & (lane == L_ - 1)
                ap = jnp.where(cond, jnp.minimum(ap, a0), ap)
            else:
                def dbody2(j, acc):
                    aps, ans = acc
                    aps, ans = list(aps), list(ans)
                    base = j * ACC_
                    for k in range(ACC_):
                        dvec = (lane + (base + k)) & (D_ - 1)
                        sv = plsc.load_gather(s_v, [dvec, row_idx])
                        pv = plsc.load_gather(rowsp_v, [row_idx, dvec])
                        nv = plsc.load_gather(rowsn_v, [row_idx, dvec])
                        dp = sv - pv
                        dn = sv - nv
                        aps[k] = aps[k] + dp * dp
                        ans[k] = ans[k] + dn * dn
                    return tuple(aps), tuple(ans)

                aps, ans = lax.fori_loop(0, D_ // ACC_, dbody2,
                                         ((zeros,) * ACC_, (zeros,) * ACC_))
                ap = sum(aps[1:], aps[0])
                an = sum(ans[1:], ans[0])
            outp_v[pl.ds(g * L_, L_)] = ap
            outn_v[pl.ds(g * L_, L_)] = an
        pltpu.sync_copy(outp_v, posq_hbm.at[b, pl.ds(t0, C_)])
        pltpu.sync_copy(outn_v, negq_hbm.at[b, pl.ds(t0, C_)])
        return carry

    lax.fori_loop(0, NCHUNK_, chunk_body, 0)


def _distances_sc(s, codes, neg, cb):
    mesh = plsc.VectorSubcoreMesh(core_axis_name="c", subcore_axis_name="s",
                                  num_cores=2, num_subcores=16)
    f = pl.kernel(
        _sc_body,
        out_type=(jax.ShapeDtypeStruct((B_, T_), jnp.float32),
                  jax.ShapeDtypeStruct((B_, T_), jnp.float32)),
        mesh=mesh,
        scratch_types=[
            pltpu.VMEM((C_,), jnp.int32),
            pltpu.VMEM((C_,), jnp.int32),
            pltpu.VMEM((D_, C_), jnp.float32),
            pltpu.VMEM((C_, D_), jnp.float32),
            pltpu.VMEM((C_, D_), jnp.float32),
            pltpu.VMEM((D_,), jnp.float32),
            pltpu.VMEM((C_,), jnp.float32),
            pltpu.VMEM((C_,), jnp.float32),
            pltpu.SemaphoreType.DMA,
            pltpu.SemaphoreType.DMA,
            pltpu.SemaphoreType.DMA,
        ],
        compiler_params=pltpu.CompilerParams(use_tc_tiling_on_sc=False,
                                             needs_layout_passes=False,
                                             disable_bounds_checks=True),
    )
    return f(s, codes, neg, cb)


def _finalize_body(pos_ref, neg_ref, len_ref, out_ref):
    p = jnp.sqrt(pos_ref[...])
    n = jnp.sqrt(neg_ref[...])
    lens = len_ref[...]
    valid = jnp.minimum((lens + (STRIDE - 1)) // STRIDE, T_)
    tcol = lax.broadcasted_iota(jnp.int32, (B_, T_), 1)
    mask = (tcol < valid).astype(jnp.float32)
    tl = jnp.maximum(p - n + MARGIN, 0.0) * mask
    loss = jnp.sum(tl) / (jnp.sum(mask) + 1e-8)
    out_ref[...] = jnp.full((1, 1), loss, dtype=jnp.float32)


_finalize_tc = functools.partial(
    pl.pallas_call,
    out_shape=jax.ShapeDtypeStruct((1, 1), jnp.float32),
)(_finalize_body)


def kernel(student_features, teacher_codes, codebook, lengths):
    codes = teacher_codes.astype(jnp.int32)
    neg = jax.random.randint(jax.random.key(42), (B_, T_), 0,
                             NUM_CODES).astype(jnp.int32)
    posq, negq = _distances_sc(student_features, codes, neg, codebook)
    out = _finalize_tc(posq, negq, lengths.reshape(B_, 1).astype(jnp.int32))
    return out[0, 0]


# speedup vs baseline: 1.4976x; 1.4976x over previous
import functools

import jax
import jax.numpy as jnp
from jax import lax
from jax.experimental import pallas as pl
from jax.experimental.pallas import tpu as pltpu
from jax.experimental.pallas import tpu_sc as plsc

B_ = 16
D_ = 256
T_ = 4096
NUM_CODES = 8192
MARGIN = 0.2
STRIDE = 320

L_ = 16
NW_ = 32
C_ = 64
PER_W = (B_ * T_) // NW_
NCHUNK_ = PER_W // C_
G_ = C_ // L_
ACC_ = 4


def _sc_body(s_hbm, codes_hbm, neg_hbm, cb_hbm, posq_hbm, negq_hbm,
             idxp_all, idxn_all, s_v0, s_v1, rp0, rp1, rn0, rn1, row0_v,
             outp_all, outn_all,
             semp0, semp1, semn0, semn1, sems0, sems1):
    s_bufs = (s_v0, s_v1)
    rp = (rp0, rp1)
    rn = (rn0, rn1)
    semp = (semp0, semp1)
    semn = (semn0, semn1)
    sems = (sems0, sems1)
    cid = lax.axis_index("c")
    sid = lax.axis_index("s")
    wid = sid * 2 + cid
    b = wid // 2
    h = wid % 2
    t_base = h * (T_ // 2)
    pltpu.sync_copy(cb_hbm.at[0], row0_v)
    pltpu.sync_copy(codes_hbm.at[b, pl.ds(t_base, PER_W)], idxp_all)
    pltpu.sync_copy(neg_hbm.at[b, pl.ds(t_base, PER_W)], idxn_all)
    lane = lax.iota(jnp.int32, L_)
    zeros = jnp.zeros((L_,), jnp.float32)

    def dmas(ci, k):
        off = ci * C_
        t0 = t_base + off
        return (
            pltpu.make_async_copy(cb_hbm.at[idxp_all.at[pl.ds(off, C_)]],
                                  rp[k], semp[k]),
            pltpu.make_async_copy(cb_hbm.at[idxn_all.at[pl.ds(off, C_)]],
                                  rn[k], semn[k]),
            pltpu.make_async_copy(s_hbm.at[b, :, pl.ds(t0, C_)],
                                  s_bufs[k], sems[k]),
        )

    def start(ci, k):
        for d in dmas(ci, k):
            d.start()

    def waitall(ci, k):
        for d in dmas(ci, k):
            d.wait()

    def compute(ci, k):
        s_v = s_bufs[k]
        rowsp_v = rp[k]
        rowsn_v = rn[k]
        off = ci * C_
        for g in range(G_):
            row_idx = lane + g * L_
            edge = g == 0 or g == G_ - 1
            if edge:
                def dbody3(j, acc):
                    aps, ans, a0s = acc
                    aps, ans, a0s = list(aps), list(ans), list(a0s)
                    base = j * ACC_
                    for k in range(ACC_):
                        dvec = (lane + (base + k)) & (D_ - 1)
                        sv = plsc.load_gather(s_v, [dvec, row_idx])
                        pv = plsc.load_gather(rowsp_v, [row_idx, dvec])
                        nv = plsc.load_gather(rowsn_v, [row_idx, dvec])
                        c0 = plsc.load_gather(row0_v, [dvec])
                        dp = sv - pv
                        dn = sv - nv
                        d0 = sv - c0
                        aps[k] = aps[k] + dp * dp
                        ans[k] = ans[k] + dn * dn
                        a0s[k] = a0s[k] + d0 * d0
                    return tuple(aps), tuple(ans), tuple(a0s)

                aps, ans, a0s = lax.fori_loop(
                    0, D_ // ACC_, dbody3,
                    ((zeros,) * ACC_, (zeros,) * ACC_, (zeros,) * ACC_))
                ap = sum(aps[1:], aps[0])
                an = sum(ans[1:], ans[0])
                a0 = sum(a0s[1:], a0s[0])
                if g == 0:
                    cond = (h == 0) & (ci == 0) & (lane == 0)
                else:
                    cond = (h == 1) & (ci == NCHUNK_ - 1) & (lane == L_ - 1)
                ap = jnp.where(cond, jnp.minimum(ap, a0), ap)
            else:
                def dbody2(j, acc):
                    aps, ans = acc
                    aps, ans = list(aps), list(ans)
                    base = j * ACC_
                    for k in range(ACC_):
                        dvec = (lane + (base + k)) & (D_ - 1)
                        sv = plsc.load_gather(s_v, [dvec, row_idx])
                        pv = plsc.load_gather(rowsp_v, [row_idx, dvec])
                        nv = plsc.load_gather(rowsn_v, [row_idx, dvec])
                        dp = sv - pv
                        dn = sv - nv
                        aps[k] = aps[k] + dp * dp
                        ans[k] = ans[k] + dn * dn
                    return tuple(aps), tuple(ans)

                aps, ans = lax.fori_loop(0, D_ // ACC_, dbody2,
                                         ((zeros,) * ACC_, (zeros,) * ACC_))
                ap = sum(aps[1:], aps[0])
                an = sum(ans[1:], ans[0])
            outp_all[pl.ds(off + g * L_, L_)] = ap
            outn_all[pl.ds(off + g * L_, L_)] = an

    start(0, 0)

    def pair(i, carry):
        c0 = 2 * i
        start(c0 + 1, 1)
        waitall(c0, 0)
        compute(c0, 0)
        start(jnp.minimum(c0 + 2, NCHUNK_ - 1), 0)
        waitall(c0 + 1, 1)
        compute(c0 + 1, 1)
        return carry

    lax.fori_loop(0, NCHUNK_ // 2, pair, 0)
    waitall(NCHUNK_ - 1, 0)
    pltpu.sync_copy(outp_all, posq_hbm.at[b, pl.ds(t_base, PER_W)])
    pltpu.sync_copy(outn_all, negq_hbm.at[b, pl.ds(t_base, PER_W)])


def _distances_sc(s, codes, neg, cb):
    mesh = plsc.VectorSubcoreMesh(core_axis_name="c", subcore_axis_name="s",
                                  num_cores=2, num_subcores=16)
    f = pl.kernel(
        _sc_body,
        out_type=(jax.ShapeDtypeStruct((B_, T_), jnp.float32),
                  jax.ShapeDtypeStruct((B_, T_), jnp.float32)),
        mesh=mesh,
        scratch_types=[
            pltpu.VMEM((PER_W,), jnp.int32),
            pltpu.VMEM((PER_W,), jnp.int32),
            pltpu.VMEM((D_, C_), jnp.float32),
            pltpu.VMEM((D_, C_), jnp.float32),
            pltpu.VMEM((C_, D_), jnp.float32),
            pltpu.VMEM((C_, D_), jnp.float32),
            pltpu.VMEM((C_, D_), jnp.float32),
            pltpu.VMEM((C_, D_), jnp.float32),
            pltpu.VMEM((D_,), jnp.float32),
            pltpu.VMEM((PER_W,), jnp.float32),
            pltpu.VMEM((PER_W,), jnp.float32),
            pltpu.SemaphoreType.DMA,
            pltpu.SemaphoreType.DMA,
            pltpu.SemaphoreType.DMA,
            pltpu.SemaphoreType.DMA,
            pltpu.SemaphoreType.DMA,
            pltpu.SemaphoreType.DMA,
        ],
        compiler_params=pltpu.CompilerParams(use_tc_tiling_on_sc=False,
                                             needs_layout_passes=False,
                                             disable_bounds_checks=True),
    )
    return f(s, codes, neg, cb)


def _finalize_body(pos_ref, neg_ref, len_ref, out_ref):
    p = jnp.sqrt(pos_ref[...])
    n = jnp.sqrt(neg_ref[...])
    lens = len_ref[...]
    valid = jnp.minimum((lens + (STRIDE - 1)) // STRIDE, T_)
    tcol = lax.broadcasted_iota(jnp.int32, (B_, T_), 1)
    mask = (tcol < valid).astype(jnp.float32)
    tl = jnp.maximum(p - n + MARGIN, 0.0) * mask
    loss = jnp.sum(tl) / (jnp.sum(mask) + 1e-8)
    out_ref[...] = jnp.full((1, 1), loss, dtype=jnp.float32)


_finalize_tc = functools.partial(
    pl.pallas_call,
    out_shape=jax.ShapeDtypeStruct((1, 1), jnp.float32),
)(_finalize_body)


def kernel(student_features, teacher_codes, codebook, lengths):
    codes = teacher_codes.astype(jnp.int32)
    neg = jax.random.randint(jax.random.key(42), (B_, T_), 0,
                             NUM_CODES).astype(jnp.int32)
    posq, negq = _distances_sc(student_features, codes, neg, codebook)
    out = _finalize_tc(posq, negq, lengths.reshape(B_, 1).astype(jnp.int32))
    return out[0, 0]


# speedup vs baseline: 1.5077x; 1.0067x over previous
import functools

import jax
import jax.numpy as jnp
from jax import lax
from jax.experimental import pallas as pl
from jax.experimental.pallas import tpu as pltpu
from jax.experimental.pallas import tpu_sc as plsc

B_ = 16
D_ = 256
T_ = 4096
NUM_CODES = 8192
MARGIN = 0.2
STRIDE = 320

L_ = 16
NW_ = 32
C_ = 64
PER_W = (B_ * T_) // NW_
NCHUNK_ = PER_W // C_
G_ = C_ // L_
ACC_ = 4


def _sc_body(s_hbm, codes_hbm, neg_hbm, cb_hbm, posq_hbm, negq_hbm,
             idxp_all, idxn_all, s_v0, s_v1, rp0, rp1, rn0, rn1, row0_v,
             outp_all, outn_all,
             semp0, semp1, semn0, semn1, sems0, sems1):
    s_bufs = (s_v0, s_v1)
    rp = (rp0, rp1)
    rn = (rn0, rn1)
    semp = (semp0, semp1)
    semn = (semn0, semn1)
    sems = (sems0, sems1)
    cid = lax.axis_index("c")
    sid = lax.axis_index("s")
    wid = sid * 2 + cid
    b = wid // 2
    h = wid % 2
    t_base = h * (T_ // 2)
    pltpu.sync_copy(cb_hbm.at[0], row0_v)
    pltpu.sync_copy(codes_hbm.at[b, pl.ds(t_base, PER_W)], idxp_all)
    pltpu.sync_copy(neg_hbm.at[b, pl.ds(t_base, PER_W)], idxn_all)
    lane = lax.iota(jnp.int32, L_)
    zeros = jnp.zeros((L_,), jnp.float32)

    def dmas(ci, k):
        off = ci * C_
        t0 = t_base + off
        return (
            pltpu.make_async_copy(cb_hbm.at[idxp_all.at[pl.ds(off, C_)]],
                                  rp[k], semp[k]),
            pltpu.make_async_copy(cb_hbm.at[idxn_all.at[pl.ds(off, C_)]],
                                  rn[k], semn[k]),
            pltpu.make_async_copy(s_hbm.at[b, :, pl.ds(t0, C_)],
                                  s_bufs[k], sems[k]),
        )

    def start(ci, k):
        for d in dmas(ci, k):
            d.start()

    def waitall(ci, k):
        for d in dmas(ci, k):
            d.wait()

    def compute(ci, k):
        s_v = s_bufs[k]
        rowsp_v = rp[k]
        rowsn_v = rn[k]
        off = ci * C_
        for g in range(G_):
            row_idx = lane + g * L_

            def dbody2(j, acc):
                aps, ans = acc
                aps, ans = list(aps), list(ans)
                base = j * ACC_
                for k in range(ACC_):
                    dvec = (lane + (base + k)) & (D_ - 1)
                    sv = plsc.load_gather(s_v, [dvec, row_idx])
                    pv = plsc.load_gather(rowsp_v, [row_idx, dvec])
                    nv = plsc.load_gather(rowsn_v, [row_idx, dvec])
                    dp = sv - pv
                    dn = sv - nv
                    aps[k] = aps[k] + dp * dp
                    ans[k] = ans[k] + dn * dn
                return tuple(aps), tuple(ans)

            aps, ans = lax.fori_loop(0, D_ // ACC_, dbody2,
                                     ((zeros,) * ACC_, (zeros,) * ACC_))
            ap = sum(aps[1:], aps[0])
            an = sum(ans[1:], ans[0])
            outp_all[pl.ds(off + g * L_, L_)] = ap
            outn_all[pl.ds(off + g * L_, L_)] = an

        def edge_fix(g, lane_pick):
            row_idx = lane + g * L_

            def d0body(j, acc):
                acc = list(acc)
                base = j * ACC_
                for k in range(ACC_):
                    dvec = (lane + (base + k)) & (D_ - 1)
                    sv = plsc.load_gather(s_v, [dvec, row_idx])
                    c0 = plsc.load_gather(row0_v, [dvec])
                    d0 = sv - c0
                    acc[k] = acc[k] + d0 * d0
                return tuple(acc)

            a0s = lax.fori_loop(0, D_ // ACC_, d0body, (zeros,) * ACC_)
            a0 = sum(a0s[1:], a0s[0])
            ap0 = outp_all[pl.ds(off + g * L_, L_)]
            newp = jnp.where(lane == lane_pick, jnp.minimum(ap0, a0), ap0)
            outp_all[pl.ds(off + g * L_, L_)] = newp

        @pl.when((h == 0) & (ci == 0))
        def _():
            edge_fix(0, 0)

        @pl.when((h == 1) & (ci == NCHUNK_ - 1))
        def _():
            edge_fix(G_ - 1, L_ - 1)

    start(0, 0)

    def pair(i, carry):
        c0 = 2 * i
        start(c0 + 1, 1)
        waitall(c0, 0)
        compute(c0, 0)
        start(jnp.minimum(c0 + 2, NCHUNK_ - 1), 0)
        waitall(c0 + 1, 1)
        compute(c0 + 1, 1)
        return carry

    lax.fori_loop(0, NCHUNK_ // 2, pair, 0)
    waitall(NCHUNK_ - 1, 0)
    pltpu.sync_copy(outp_all, posq_hbm.at[b, pl.ds(t_base, PER_W)])
    pltpu.sync_copy(outn_all, negq_hbm.at[b, pl.ds(t_base, PER_W)])


def _distances_sc(s, codes, neg, cb):
    mesh = plsc.VectorSubcoreMesh(core_axis_name="c", subcore_axis_name="s",
                                  num_cores=2, num_subcores=16)
    f = pl.kernel(
        _sc_body,
        out_type=(jax.ShapeDtypeStruct((B_, T_), jnp.float32),
                  jax.ShapeDtypeStruct((B_, T_), jnp.float32)),
        mesh=mesh,
        scratch_types=[
            pltpu.VMEM((PER_W,), jnp.int32),
            pltpu.VMEM((PER_W,), jnp.int32),
            pltpu.VMEM((D_, C_), jnp.float32),
            pltpu.VMEM((D_, C_), jnp.float32),
            pltpu.VMEM((C_, D_), jnp.float32),
            pltpu.VMEM((C_, D_), jnp.float32),
            pltpu.VMEM((C_, D_), jnp.float32),
            pltpu.VMEM((C_, D_), jnp.float32),
            pltpu.VMEM((D_,), jnp.float32),
            pltpu.VMEM((PER_W,), jnp.float32),
            pltpu.VMEM((PER_W,), jnp.float32),
            pltpu.SemaphoreType.DMA,
            pltpu.SemaphoreType.DMA,
            pltpu.SemaphoreType.DMA,
            pltpu.SemaphoreType.DMA,
            pltpu.SemaphoreType.DMA,
            pltpu.SemaphoreType.DMA,
        ],
        compiler_params=pltpu.CompilerParams(use_tc_tiling_on_sc=False,
                                             needs_layout_passes=False,
                                             disable_bounds_checks=True),
    )
    return f(s, codes, neg, cb)


def _finalize_body(pos_ref, neg_ref, len_ref, out_ref):
    p = jnp.sqrt(pos_ref[...])
    n = jnp.sqrt(neg_ref[...])
    lens = len_ref[...]
    valid = jnp.minimum((lens + (STRIDE - 1)) // STRIDE, T_)
    tcol = lax.broadcasted_iota(jnp.int32, (B_, T_), 1)
    mask = (tcol < valid).astype(jnp.float32)
    tl = jnp.maximum(p - n + MARGIN, 0.0) * mask
    loss = jnp.sum(tl) / (jnp.sum(mask) + 1e-8)
    out_ref[...] = jnp.full((1, 1), loss, dtype=jnp.float32)


_finalize_tc = functools.partial(
    pl.pallas_call,
    out_shape=jax.ShapeDtypeStruct((1, 1), jnp.float32),
)(_finalize_body)


def kernel(student_features, teacher_codes, codebook, lengths):
    codes = teacher_codes.astype(jnp.int32)
    neg = jax.random.randint(jax.random.key(42), (B_, T_), 0,
                             NUM_CODES).astype(jnp.int32)
    posq, negq = _distances_sc(student_features, codes, neg, codebook)
    out = _finalize_tc(posq, negq, lengths.reshape(B_, 1).astype(jnp.int32))
    return out[0, 0]


# speedup vs baseline: 2.0997x; 1.3927x over previous
import functools

import jax
import jax.numpy as jnp
from jax import lax
from jax.experimental import pallas as pl
from jax.experimental.pallas import tpu as pltpu
from jax.experimental.pallas import tpu_sc as plsc

B_ = 16
D_ = 256
T_ = 4096
NUM_CODES = 8192
MARGIN = 0.2
STRIDE = 320

L_ = 16
NW_ = 32
C_ = 64
PER_W = (B_ * T_) // NW_
NCHUNK_ = PER_W // C_
G_ = C_ // L_
ACC_ = 4


def _sc_body(s_hbm, codes_hbm, neg_hbm, cb_hbm, posq_hbm, negq_hbm,
             idxp_all, idxn_all, s_v0, s_v1, rp0, rp1, rn0, rn1, row0_v,
             outp_all, outn_all,
             semp0, semp1, semn0, semn1, sems0, sems1):
    s_bufs = (s_v0, s_v1)
    rp = (rp0, rp1)
    rn = (rn0, rn1)
    semp = (semp0, semp1)
    semn = (semn0, semn1)
    sems = (sems0, sems1)
    cid = lax.axis_index("c")
    sid = lax.axis_index("s")
    wid = sid * 2 + cid
    b = wid // 2
    h = wid % 2
    t_base = h * (T_ // 2)
    pltpu.sync_copy(cb_hbm.at[0], row0_v)
    pltpu.sync_copy(codes_hbm.at[b, pl.ds(t_base, PER_W)], idxp_all)
    pltpu.sync_copy(neg_hbm.at[b, pl.ds(t_base, PER_W)], idxn_all)
    lane = lax.iota(jnp.int32, L_)
    zeros = jnp.zeros((L_,), jnp.float32)

    def dmas(ci, k):
        off = ci * C_
        t0 = t_base + off
        return (
            pltpu.make_async_copy(cb_hbm.at[idxp_all.at[pl.ds(off, C_)]],
                                  rp[k], semp[k]),
            pltpu.make_async_copy(cb_hbm.at[idxn_all.at[pl.ds(off, C_)]],
                                  rn[k], semn[k]),
            pltpu.make_async_copy(
                s_hbm.at[b, :, t0 // 128, :, pl.ds(t0 % 128, C_)],
                s_bufs[k], sems[k]),
        )

    def start(ci, k):
        for d in dmas(ci, k):
            d.start()

    def waitall(ci, k):
        for d in dmas(ci, k):
            d.wait()

    def compute(ci, k):
        s_v = s_bufs[k]
        rowsp_v = rp[k]
        rowsn_v = rn[k]
        off = ci * C_
        for g in range(G_):
            row_idx = lane + g * L_

            def dbody2(j, acc):
                aps, ans = acc
                aps, ans = list(aps), list(ans)
                base = j * ACC_
                for k in range(ACC_):
                    dvec = (lane + (base + k)) & (D_ - 1)
                    sv = plsc.load_gather(
                        s_v, [dvec >> 3, dvec & 7, row_idx])
                    pv = plsc.load_gather(rowsp_v, [row_idx, dvec])
                    nv = plsc.load_gather(rowsn_v, [row_idx, dvec])
                    dp = sv - pv
                    dn = sv - nv
                    aps[k] = aps[k] + dp * dp
                    ans[k] = ans[k] + dn * dn
                return tuple(aps), tuple(ans)

            aps, ans = lax.fori_loop(0, D_ // ACC_, dbody2,
                                     ((zeros,) * ACC_, (zeros,) * ACC_))
            ap = sum(aps[1:], aps[0])
            an = sum(ans[1:], ans[0])
            outp_all[pl.ds(off + g * L_, L_)] = ap
            outn_all[pl.ds(off + g * L_, L_)] = an

        def edge_fix(g, lane_pick):
            row_idx = lane + g * L_

            def d0body(j, acc):
                acc = list(acc)
                base = j * ACC_
                for k in range(ACC_):
                    dvec = (lane + (base + k)) & (D_ - 1)
                    sv = plsc.load_gather(
                        s_v, [dvec >> 3, dvec & 7, row_idx])
                    c0 = plsc.load_gather(row0_v, [dvec])
                    d0 = sv - c0
                    acc[k] = acc[k] + d0 * d0
                return tuple(acc)

            a0s = lax.fori_loop(0, D_ // ACC_, d0body, (zeros,) * ACC_)
            a0 = sum(a0s[1:], a0s[0])
            ap0 = outp_all[pl.ds(off + g * L_, L_)]
            newp = jnp.where(lane == lane_pick, jnp.minimum(ap0, a0), ap0)
            outp_all[pl.ds(off + g * L_, L_)] = newp

        @pl.when((h == 0) & (ci == 0))
        def _():
            edge_fix(0, 0)

        @pl.when((h == 1) & (ci == NCHUNK_ - 1))
        def _():
            edge_fix(G_ - 1, L_ - 1)

    start(0, 0)

    def pair(i, carry):
        c0 = 2 * i
        start(c0 + 1, 1)
        waitall(c0, 0)
        compute(c0, 0)
        start(jnp.minimum(c0 + 2, NCHUNK_ - 1), 0)
        waitall(c0 + 1, 1)
        compute(c0 + 1, 1)
        return carry

    lax.fori_loop(0, NCHUNK_ // 2, pair, 0)
    waitall(NCHUNK_ - 1, 0)
    pltpu.sync_copy(outp_all, posq_hbm.at[b, pl.ds(t_base, PER_W)])
    pltpu.sync_copy(outn_all, negq_hbm.at[b, pl.ds(t_base, PER_W)])


def _distances_sc(s, codes, neg, cb):
    mesh = plsc.VectorSubcoreMesh(core_axis_name="c", subcore_axis_name="s",
                                  num_cores=2, num_subcores=16)
    f = pl.kernel(
        _sc_body,
        out_type=(jax.ShapeDtypeStruct((B_, T_), jnp.float32),
                  jax.ShapeDtypeStruct((B_, T_), jnp.float32)),
        mesh=mesh,
        scratch_types=[
            pltpu.VMEM((PER_W,), jnp.int32),
            pltpu.VMEM((PER_W,), jnp.int32),
            pltpu.VMEM((D_ // 8, 8, C_), jnp.float32),
            pltpu.VMEM((D_ // 8, 8, C_), jnp.float32),
            pltpu.VMEM((C_, D_), jnp.float32),
            pltpu.VMEM((C_, D_), jnp.float32),
            pltpu.VMEM((C_, D_), jnp.float32),
            pltpu.VMEM((C_, D_), jnp.float32),
            pltpu.VMEM((D_,), jnp.float32),
            pltpu.VMEM((PER_W,), jnp.float32),
            pltpu.VMEM((PER_W,), jnp.float32),
            pltpu.SemaphoreType.DMA,
            pltpu.SemaphoreType.DMA,
            pltpu.SemaphoreType.DMA,
            pltpu.SemaphoreType.DMA,
            pltpu.SemaphoreType.DMA,
            pltpu.SemaphoreType.DMA,
        ],
        compiler_params=pltpu.CompilerParams(use_tc_tiling_on_sc=False,
                                             needs_layout_passes=False,
                                             disable_bounds_checks=True),
    )
    return f(s, codes, neg, cb)


def _finalize_body(pos_ref, neg_ref, len_ref, out_ref):
    p = jnp.sqrt(pos_ref[...])
    n = jnp.sqrt(neg_ref[...])
    lens = len_ref[...]
    valid = jnp.minimum((lens + (STRIDE - 1)) // STRIDE, T_)
    tcol = lax.broadcasted_iota(jnp.int32, (B_, T_), 1)
    mask = (tcol < valid).astype(jnp.float32)
    tl = jnp.maximum(p - n + MARGIN, 0.0) * mask
    loss = jnp.sum(tl) / (jnp.sum(mask) + 1e-8)
    out_ref[...] = jnp.full((1, 1), loss, dtype=jnp.float32)


_finalize_tc = functools.partial(
    pl.pallas_call,
    out_shape=jax.ShapeDtypeStruct((1, 1), jnp.float32),
)(_finalize_body)


def kernel(student_features, teacher_codes, codebook, lengths):
    codes = teacher_codes.astype(jnp.int32)
    neg = jax.random.randint(jax.random.key(42), (B_, T_), 0,
                             NUM_CODES).astype(jnp.int32)
    s5 = student_features.reshape(B_, D_ // 8, 8, T_ // 128, 128)
    s5 = s5.transpose(0, 1, 3, 2, 4)
    posq, negq = _distances_sc(s5, codes, neg, codebook)
    out = _finalize_tc(posq, negq, lengths.reshape(B_, 1).astype(jnp.int32))
    return out[0, 0]


# speedup vs baseline: 2.1124x; 1.0061x over previous
import functools

import jax
import jax.numpy as jnp
from jax import lax
from jax.experimental import pallas as pl
from jax.experimental.pallas import tpu as pltpu
from jax.experimental.pallas import tpu_sc as plsc

B_ = 16
D_ = 256
T_ = 4096
NUM_CODES = 8192
MARGIN = 0.2
STRIDE = 320

L_ = 16
NW_ = 32
C_ = 64
PER_W = (B_ * T_) // NW_
NCHUNK_ = PER_W // C_
G_ = C_ // L_
ACC_ = 4


def _sc_body(s_hbm, codes_hbm, neg_hbm, cb_hbm, posq_hbm, negq_hbm,
             idxp_all, idxn_all, s_v0, s_v1, rp0, rp1, rn0, rn1, row0_v,
             outp_all, outn_all,
             semp0, semp1, semn0, semn1, sems0, sems1):
    s_bufs = (s_v0, s_v1)
    rp = (rp0, rp1)
    rn = (rn0, rn1)
    semp = (semp0, semp1)
    semn = (semn0, semn1)
    sems = (sems0, sems1)
    cid = lax.axis_index("c")
    sid = lax.axis_index("s")
    wid = sid * 2 + cid
    b = wid // 2
    h = wid % 2
    t_base = h * (T_ // 2)
    stage = (
        pltpu.make_async_copy(cb_hbm.at[0], row0_v, semp0),
        pltpu.make_async_copy(codes_hbm.at[b, pl.ds(t_base, PER_W)],
                              idxp_all, semn0),
        pltpu.make_async_copy(neg_hbm.at[b, pl.ds(t_base, PER_W)],
                              idxn_all, sems0),
    )
    for cp in stage:
        cp.start()
    for cp in stage:
        cp.wait()
    lane = lax.iota(jnp.int32, L_)
    zeros = jnp.zeros((L_,), jnp.float32)

    def dmas(ci, k):
        off = ci * C_
        t0 = t_base + off
        return (
            pltpu.make_async_copy(cb_hbm.at[idxp_all.at[pl.ds(off, C_)]],
                                  rp[k], semp[k]),
            pltpu.make_async_copy(cb_hbm.at[idxn_all.at[pl.ds(off, C_)]],
                                  rn[k], semn[k]),
            pltpu.make_async_copy(
                s_hbm.at[b, :, t0 // 128, :, pl.ds(t0 % 128, C_)],
                s_bufs[k], sems[k]),
        )

    def start(ci, k):
        for d in dmas(ci, k):
            d.start()

    def waitall(ci, k):
        for d in dmas(ci, k):
            d.wait()

    def compute(ci, k):
        s_v = s_bufs[k]
        rowsp_v = rp[k]
        rowsn_v = rn[k]
        off = ci * C_
        for g in range(G_):
            row_idx = lane + g * L_

            def dbody2(j, acc):
                aps, ans = acc
                aps, ans = list(aps), list(ans)
                base = j * ACC_
                for k in range(ACC_):
                    dvec = (lane + (base + k)) & (D_ - 1)
                    sv = plsc.load_gather(
                        s_v, [dvec >> 3, dvec & 7, row_idx])
                    pv = plsc.load_gather(rowsp_v, [row_idx, dvec])
                    nv = plsc.load_gather(rowsn_v, [row_idx, dvec])
                    dp = sv - pv
                    dn = sv - nv
                    aps[k] = aps[k] + dp * dp
                    ans[k] = ans[k] + dn * dn
                return tuple(aps), tuple(ans)

            aps, ans = lax.fori_loop(0, D_ // ACC_, dbody2,
                                     ((zeros,) * ACC_, (zeros,) * ACC_))
            ap = sum(aps[1:], aps[0])
            an = sum(ans[1:], ans[0])
            outp_all[pl.ds(off + g * L_, L_)] = ap
            outn_all[pl.ds(off + g * L_, L_)] = an

        def edge_fix(g, lane_pick):
            row_idx = lane + g * L_

            def d0body(j, acc):
                acc = list(acc)
                base = j * ACC_
                for k in range(ACC_):
                    dvec = (lane + (base + k)) & (D_ - 1)
                    sv = plsc.load_gather(
                        s_v, [dvec >> 3, dvec & 7, row_idx])
                    c0 = plsc.load_gather(row0_v, [dvec])
                    d0 = sv - c0
                    acc[k] = acc[k] + d0 * d0
                return tuple(acc)

            a0s = lax.fori_loop(0, D_ // ACC_, d0body, (zeros,) * ACC_)
            a0 = sum(a0s[1:], a0s[0])
            ap0 = outp_all[pl.ds(off + g * L_, L_)]
            newp = jnp.where(lane == lane_pick, jnp.minimum(ap0, a0), ap0)
            outp_all[pl.ds(off + g * L_, L_)] = newp

        @pl.when((h == 0) & (ci == 0))
        def _():
            edge_fix(0, 0)

        @pl.when((h == 1) & (ci == NCHUNK_ - 1))
        def _():
            edge_fix(G_ - 1, L_ - 1)

    start(0, 0)

    def pair(i, carry):
        c0 = 2 * i
        start(c0 + 1, 1)
        waitall(c0, 0)
        compute(c0, 0)
        start(jnp.minimum(c0 + 2, NCHUNK_ - 1), 0)
        waitall(c0 + 1, 1)
        compute(c0 + 1, 1)
        return carry

    lax.fori_loop(0, NCHUNK_ // 2, pair, 0)
    waitall(NCHUNK_ - 1, 0)
    pltpu.sync_copy(outp_all, posq_hbm.at[b, pl.ds(t_base, PER_W)])
    pltpu.sync_copy(outn_all, negq_hbm.at[b, pl.ds(t_base, PER_W)])


def _distances_sc(s, codes, neg, cb):
    mesh = plsc.VectorSubcoreMesh(core_axis_name="c", subcore_axis_name="s",
                                  num_cores=2, num_subcores=16)
    f = pl.kernel(
        _sc_body,
        out_type=(jax.ShapeDtypeStruct((B_, T_), jnp.float32),
                  jax.ShapeDtypeStruct((B_, T_), jnp.float32)),
        mesh=mesh,
        scratch_types=[
            pltpu.VMEM((PER_W,), jnp.int32),
            pltpu.VMEM((PER_W,), jnp.int32),
            pltpu.VMEM((D_ // 8, 8, C_), jnp.float32),
            pltpu.VMEM((D_ // 8, 8, C_), jnp.float32),
            pltpu.VMEM((C_, D_), jnp.float32),
            pltpu.VMEM((C_, D_), jnp.float32),
            pltpu.VMEM((C_, D_), jnp.float32),
            pltpu.VMEM((C_, D_), jnp.float32),
            pltpu.VMEM((D_,), jnp.float32),
            pltpu.VMEM((PER_W,), jnp.float32),
            pltpu.VMEM((PER_W,), jnp.float32),
            pltpu.SemaphoreType.DMA,
            pltpu.SemaphoreType.DMA,
            pltpu.SemaphoreType.DMA,
            pltpu.SemaphoreType.DMA,
            pltpu.SemaphoreType.DMA,
            pltpu.SemaphoreType.DMA,
        ],
        compiler_params=pltpu.CompilerParams(use_tc_tiling_on_sc=False,
                                             needs_layout_passes=False,
                                             disable_bounds_checks=True),
    )
    return f(s, codes, neg, cb)


def _finalize_body(pos_ref, neg_ref, len_ref, out_ref):
    p = jnp.sqrt(pos_ref[...])
    n = jnp.sqrt(neg_ref[...])
    lens = len_ref[...]
    valid = jnp.minimum((lens + (STRIDE - 1)) // STRIDE, T_)
    tcol = lax.broadcasted_iota(jnp.int32, (B_, T_), 1)
    mask = (tcol < valid).astype(jnp.float32)
    tl = jnp.maximum(p - n + MARGIN, 0.0) * mask
    loss = jnp.sum(tl) / (jnp.sum(mask) + 1e-8)
    out_ref[...] = jnp.full((1, 1), loss, dtype=jnp.float32)


_finalize_tc = functools.partial(
    pl.pallas_call,
    out_shape=jax.ShapeDtypeStruct((1, 1), jnp.float32),
)(_finalize_body)


def kernel(student_features, teacher_codes, codebook, lengths):
    codes = teacher_codes.astype(jnp.int32)
    neg = jax.random.randint(jax.random.key(42), (B_, T_), 0,
                             NUM_CODES).astype(jnp.int32)
    s5 = student_features.reshape(B_, D_ // 8, 8, T_ // 128, 128)
    s5 = s5.transpose(0, 1, 3, 2, 4)
    posq, negq = _distances_sc(s5, codes, neg, codebook)
    out = _finalize_tc(posq, negq, lengths.reshape(B_, 1).astype(jnp.int32))
    return out[0, 0]
